# baseline (speedup 1.0000x reference)
import jax
import jax.numpy as jnp
from jax.experimental import pallas as pl
from jax.experimental.pallas import tpu as pltpu

_N = 64
_R = 4
_C = 16
_DI = 8
_DB = 2
_DSI = 8
_DSJ = 64

_INTERPRET = False


def _dot_tn(a, b):
    return jax.lax.dot_general(a, b, (((0,), (0,)), ((), ())),
                               preferred_element_type=jnp.float32)


def _dot_nn(a, b):
    return jax.lax.dot_general(a, b, (((1,), (0,)), ((), ())),
                               preferred_element_type=jnp.float32)


def _col(b_ref):
    return jnp.swapaxes(b_ref[...], 0, 1)


def _mlp_body(x_ref, w1_ref, b1_ref, w2_ref, b2_ref, w3_ref, b3_ref,
              wx_ref, bx_ref, g_ref, h_ref, xout_ref):
    xt = jnp.swapaxes(x_ref[...], 0, 1)
    h = jnp.tanh(_dot_tn(w1_ref[...], xt) + _col(b1_ref))
    h = jnp.tanh(_dot_tn(w2_ref[...], h) + _col(b2_ref))
    h = jnp.tanh(_dot_tn(w3_ref[...], h) + _col(b3_ref))
    h_ref[...] = h
    xl = _dot_tn(wx_ref[...], h) + _col(bx_ref)
    m = jnp.max(xl, axis=0, keepdims=True)
    e = jnp.exp(xl - m)
    s = _dot_tn(g_ref[...], e)
    sb = _dot_nn(g_ref[...], s)
    xout_ref[...] = e / sb


def _logits_body(h_ref, wa_ref, ba_ref, out_ref):
    at = _dot_tn(wa_ref[...], h_ref[...]) + _col(ba_ref)
    b = at.shape[-1]
    out_ref[...] = at.reshape(_DI, _N, _R, b)


def _sym_body(d_ref, m_ref, out_ref):
    d = d_ref[...]
    m = jnp.swapaxes(m_ref[...], 0, 1)
    v = 0.5 * (d + m)
    v0 = v[:, :, 0:1, :]
    v1 = v[:, :, 1:2, :]
    v2 = v[:, :, 2:3, :]
    v3 = v[:, :, 3:4, :]
    mx = jnp.maximum(jnp.maximum(v0, v1), jnp.maximum(v2, v3))
    e = jnp.exp(v - mx)
    s = (e[:, :, 0:1, :] + e[:, :, 1:2, :]
         + e[:, :, 2:3, :] + e[:, :, 3:4, :])
    out_ref[...] = e / s


def kernel(input, W1, b1, W2, b2, W3, b3, Wx, bx, Wa, ba):
    B, Z = input.shape
    H1 = W1.shape[1]
    H2 = W2.shape[1]
    H3 = W3.shape[1]
    NX = Wx.shape[1]
    NA = Wa.shape[1]

    gidx = jnp.arange(NX) // _C
    Gx = (gidx[:, None] == jnp.arange(_N)[None, :]).astype(jnp.float32)

    hT, xT = pl.pallas_call(
        _mlp_body,
        grid=(1,),
        in_specs=[
            pl.BlockSpec((B, Z), lambda i: (0, 0)),
            pl.BlockSpec((Z, H1), lambda i: (0, 0)),
            pl.BlockSpec((1, H1), lambda i: (0, 0)),
            pl.BlockSpec((H1, H2), lambda i: (0, 0)),
            pl.BlockSpec((1, H2), lambda i: (0, 0)),
            pl.BlockSpec((H2, H3), lambda i: (0, 0)),
            pl.BlockSpec((1, H3), lambda i: (0, 0)),
            pl.BlockSpec((H3, NX), lambda i: (0, 0)),
            pl.BlockSpec((1, NX), lambda i: (0, 0)),
            pl.BlockSpec((NX, _N), lambda i: (0, 0)),
        ],
        out_specs=[
            pl.BlockSpec((H3, B), lambda i: (0, 0)),
            pl.BlockSpec((NX, B), lambda i: (0, 0)),
        ],
        out_shape=[
            jax.ShapeDtypeStruct((H3, B), jnp.float32),
            jax.ShapeDtypeStruct((NX, B), jnp.float32),
        ],
        compiler_params=pltpu.CompilerParams(
            dimension_semantics=("arbitrary",)),
        interpret=_INTERPRET,
    )(input, W1, b1[None], W2, b2[None], W3, b3[None],
      Wx, bx[None], Gx)

    ct = _DI * _N * _R
    aT = pl.pallas_call(
        _logits_body,
        grid=(_N // _DI,),
        in_specs=[
            pl.BlockSpec((H3, B), lambda i: (0, 0)),
            pl.BlockSpec((H3, ct), lambda i: (0, i)),
            pl.BlockSpec((1, ct), lambda i: (0, i)),
        ],
        out_specs=pl.BlockSpec((_DI, _N, _R, B), lambda i: (i, 0, 0, 0)),
        out_shape=jax.ShapeDtypeStruct((_N, _N, _R, B), jnp.float32),
        compiler_params=pltpu.CompilerParams(
            dimension_semantics=("arbitrary",)),
        interpret=_INTERPRET,
    )(hT, Wa, ba[None])

    adjT = pl.pallas_call(
        _sym_body,
        grid=(_N // _DSI, _N // _DSJ),
        in_specs=[
            pl.BlockSpec((_DSI, _DSJ, _R, B), lambda i, j: (i, j, 0, 0)),
            pl.BlockSpec((_DSJ, _DSI, _R, B), lambda i, j: (j, i, 0, 0)),
        ],
        out_specs=pl.BlockSpec((_DSI, _DSJ, _R, B), lambda i, j: (i, j, 0, 0)),
        out_shape=jax.ShapeDtypeStruct((_N, _N, _R, B), jnp.float32),
        compiler_params=pltpu.CompilerParams(
            dimension_semantics=("arbitrary", "arbitrary")),
        interpret=_INTERPRET,
    )(aT, aT)

    x = jnp.transpose(xT.reshape(_N, _C, B), (2, 0, 1))
    adj = jnp.transpose(adjT, (3, 0, 1, 2))
    return x, adj


# speedup vs baseline: 1.1280x; 1.1280x over previous
import jax
import jax.numpy as jnp
from jax.experimental import pallas as pl
from jax.experimental.pallas import tpu as pltpu

_N = 64
_R = 4
_C = 16
_DI = 8
_DB = 2
_DSI = 8
_DSJ = 64

_INTERPRET = False


def _dot_tn(a, b):
    return jax.lax.dot_general(a, b, (((0,), (0,)), ((), ())),
                               preferred_element_type=jnp.float32)


def _dot_nn(a, b):
    return jax.lax.dot_general(a, b, (((1,), (0,)), ((), ())),
                               preferred_element_type=jnp.float32)


def _col(b_ref):
    return jnp.swapaxes(b_ref[...], 0, 1)


def _mlp_body(x_ref, w1_ref, b1_ref, w2_ref, b2_ref, w3_ref, b3_ref,
              wx_ref, bx_ref, g_ref, h_ref, xout_ref):
    xt = jnp.swapaxes(x_ref[...], 0, 1)
    h = jnp.tanh(_dot_tn(w1_ref[...], xt) + _col(b1_ref))
    h = jnp.tanh(_dot_tn(w2_ref[...], h) + _col(b2_ref))
    h = jnp.tanh(_dot_tn(w3_ref[...], h) + _col(b3_ref))
    h_ref[...] = h
    xl = _dot_tn(wx_ref[...], h) + _col(bx_ref)
    m = jnp.max(xl, axis=0, keepdims=True)
    e = jnp.exp(xl - m)
    s = _dot_tn(g_ref[...], e)
    sb = _dot_nn(g_ref[...], s)
    xout_ref[...] = e / sb


def _adj_body(h_ref, wa_ref, ba_ref, out_ref, acc_ref):
    s = pl.program_id(1)
    bs = h_ref.shape[-1]

    @pl.when(s < _N // _DI)
    def _compute():
        at = _dot_tn(wa_ref[...], h_ref[...]) + _col(ba_ref)
        acc_ref[pl.ds(s * _DI, _DI)] = at.reshape(_DI, _N, _R, bs)

    @pl.when(s >= _N // _DI)
    def _emit():
        it = s - _N // _DI
        d = acc_ref[pl.ds(it * _DI, _DI)]
        m = acc_ref[:, pl.ds(it * _DI, _DI)]
        v = 0.5 * (d + jnp.swapaxes(m, 0, 1))
        v0 = v[:, :, 0:1, :]
        v1 = v[:, :, 1:2, :]
        v2 = v[:, :, 2:3, :]
        v3 = v[:, :, 3:4, :]
        mx = jnp.maximum(jnp.maximum(v0, v1), jnp.maximum(v2, v3))
        e = jnp.exp(v - mx)
        z = (e[:, :, 0:1, :] + e[:, :, 1:2, :]
             + e[:, :, 2:3, :] + e[:, :, 3:4, :])
        out_ref[...] = e / z


def kernel(input, W1, b1, W2, b2, W3, b3, Wx, bx, Wa, ba):
    B, Z = input.shape
    H1 = W1.shape[1]
    H2 = W2.shape[1]
    H3 = W3.shape[1]
    NX = Wx.shape[1]
    NA = Wa.shape[1]

    gidx = jnp.arange(NX) // _C
    Gx = (gidx[:, None] == jnp.arange(_N)[None, :]).astype(jnp.float32)

    hT, xT = pl.pallas_call(
        _mlp_body,
        grid=(1,),
        in_specs=[
            pl.BlockSpec((B, Z), lambda i: (0, 0)),
            pl.BlockSpec((Z, H1), lambda i: (0, 0)),
            pl.BlockSpec((1, H1), lambda i: (0, 0)),
            pl.BlockSpec((H1, H2), lambda i: (0, 0)),
            pl.BlockSpec((1, H2), lambda i: (0, 0)),
            pl.BlockSpec((H2, H3), lambda i: (0, 0)),
            pl.BlockSpec((1, H3), lambda i: (0, 0)),
            pl.BlockSpec((H3, NX), lambda i: (0, 0)),
            pl.BlockSpec((1, NX), lambda i: (0, 0)),
            pl.BlockSpec((NX, _N), lambda i: (0, 0)),
        ],
        out_specs=[
            pl.BlockSpec((H3, B), lambda i: (0, 0)),
            pl.BlockSpec((NX, B), lambda i: (0, 0)),
        ],
        out_shape=[
            jax.ShapeDtypeStruct((H3, B), jnp.float32),
            jax.ShapeDtypeStruct((NX, B), jnp.float32),
        ],
        compiler_params=pltpu.CompilerParams(
            dimension_semantics=("arbitrary",)),
        interpret=_INTERPRET,
    )(input, W1, b1[None], W2, b2[None], W3, b3[None],
      Wx, bx[None], Gx)

    ct = _DI * _N * _R
    ns = _N // _DI
    bs = B // _DB

    def _wa_idx(bh, s):
        return (0, jnp.where(s < ns, s, 0))

    adjT = pl.pallas_call(
        _adj_body,
        grid=(_DB, 2 * ns),
        in_specs=[
            pl.BlockSpec((H3, bs), lambda bh, s: (0, bh)),
            pl.BlockSpec((H3, ct), _wa_idx),
            pl.BlockSpec((1, ct), _wa_idx),
        ],
        out_specs=pl.BlockSpec(
            (_DI, _N, _R, bs),
            lambda bh, s: (jnp.where(s < ns, 0, s - ns), 0, 0, bh)),
        out_shape=jax.ShapeDtypeStruct((_N, _N, _R, B), jnp.float32),
        scratch_shapes=[pltpu.VMEM((_N, _N, _R, bs), jnp.float32)],
        compiler_params=pltpu.CompilerParams(
            dimension_semantics=("arbitrary", "arbitrary")),
        interpret=_INTERPRET,
    )(hT, Wa, ba[None])

    x = jnp.transpose(xT.reshape(_N, _C, B), (2, 0, 1))
    adj = jnp.transpose(adjT, (3, 0, 1, 2))
    return x, adj


# speedup vs baseline: 1.4007x; 1.2417x over previous
import jax
import jax.numpy as jnp
from jax.experimental import pallas as pl
from jax.experimental.pallas import tpu as pltpu

_N = 64
_R = 4
_C = 16
_DI = 8
_DB = 2

_INTERPRET = False


def _dot_tn(a, b):
    return jax.lax.dot_general(a, b, (((0,), (0,)), ((), ())),
                               preferred_element_type=jnp.float32)


def _dot_nn(a, b):
    return jax.lax.dot_general(a, b, (((1,), (0,)), ((), ())),
                               preferred_element_type=jnp.float32)


def _col(b_ref):
    return jnp.swapaxes(b_ref[...], 0, 1)


def _mlp_body(x_ref, w1_ref, b1_ref, w2_ref, b2_ref, w3_ref, b3_ref,
              wx_ref, bx_ref, g_ref, h_ref, xout_ref):
    xt = jnp.swapaxes(x_ref[...], 0, 1)
    h = jnp.tanh(_dot_tn(w1_ref[...], xt) + _col(b1_ref))
    h = jnp.tanh(_dot_tn(w2_ref[...], h) + _col(b2_ref))
    h = jnp.tanh(_dot_tn(w3_ref[...], h) + _col(b3_ref))
    h_ref[...] = h
    xl = _dot_tn(wx_ref[...], h) + _col(bx_ref)
    m = jnp.max(xl, axis=0, keepdims=True)
    e = jnp.exp(xl - m)
    s = _dot_tn(g_ref[...], e)
    sb = _dot_nn(g_ref[...], s)
    xout_ref[...] = e / sb


def _adj_body(h_ref, wa_ref, ba_ref, out_ref, acc_ref):
    s = pl.program_id(1)
    bs = h_ref.shape[-1]

    @pl.when(s < _N // _DI)
    def _compute():
        wa16 = wa_ref[...].astype(jnp.bfloat16)
        h16 = h_ref[...].astype(jnp.bfloat16)
        at = _dot_tn(wa16, h16) + _col(ba_ref)
        acc_ref[pl.ds(s * _DI, _DI)] = at.reshape(_DI, _N, _R, bs)

    @pl.when(s >= _N // _DI)
    def _emit():
        it = s - _N // _DI
        d = acc_ref[pl.ds(it * _DI, _DI)]
        m = acc_ref[:, pl.ds(it * _DI, _DI)]
        v = 0.5 * (d + jnp.swapaxes(m, 0, 1))
        e = jnp.exp(v)
        z = (e[:, :, 0:1, :] + e[:, :, 1:2, :]
             + e[:, :, 2:3, :] + e[:, :, 3:4, :])
        out_ref[...] = e / z


def kernel(input, W1, b1, W2, b2, W3, b3, Wx, bx, Wa, ba):
    B, Z = input.shape
    H1 = W1.shape[1]
    H2 = W2.shape[1]
    H3 = W3.shape[1]
    NX = Wx.shape[1]
    NA = Wa.shape[1]

    gidx = jnp.arange(NX) // _C
    Gx = (gidx[:, None] == jnp.arange(_N)[None, :]).astype(jnp.float32)

    hT, xT = pl.pallas_call(
        _mlp_body,
        grid=(1,),
        in_specs=[
            pl.BlockSpec((B, Z), lambda i: (0, 0)),
            pl.BlockSpec((Z, H1), lambda i: (0, 0)),
            pl.BlockSpec((1, H1), lambda i: (0, 0)),
            pl.BlockSpec((H1, H2), lambda i: (0, 0)),
            pl.BlockSpec((1, H2), lambda i: (0, 0)),
            pl.BlockSpec((H2, H3), lambda i: (0, 0)),
            pl.BlockSpec((1, H3), lambda i: (0, 0)),
            pl.BlockSpec((H3, NX), lambda i: (0, 0)),
            pl.BlockSpec((1, NX), lambda i: (0, 0)),
            pl.BlockSpec((NX, _N), lambda i: (0, 0)),
        ],
        out_specs=[
            pl.BlockSpec((H3, B), lambda i: (0, 0)),
            pl.BlockSpec((NX, B), lambda i: (0, 0)),
        ],
        out_shape=[
            jax.ShapeDtypeStruct((H3, B), jnp.float32),
            jax.ShapeDtypeStruct((NX, B), jnp.float32),
        ],
        compiler_params=pltpu.CompilerParams(
            dimension_semantics=("arbitrary",)),
        interpret=_INTERPRET,
    )(input, W1, b1[None], W2, b2[None], W3, b3[None],
      Wx, bx[None], Gx)

    ct = _DI * _N * _R
    ns = _N // _DI
    bs = B // _DB

    def _wa_idx(bh, s):
        return (0, jnp.where(s < ns, s, 0))

    adjT = pl.pallas_call(
        _adj_body,
        grid=(_DB, 2 * ns),
        in_specs=[
            pl.BlockSpec((H3, bs), lambda bh, s: (0, bh)),
            pl.BlockSpec((H3, ct), _wa_idx),
            pl.BlockSpec((1, ct), _wa_idx),
        ],
        out_specs=pl.BlockSpec(
            (_DI, _N, _R, bs),
            lambda bh, s: (jnp.where(s < ns, 0, s - ns), 0, 0, bh)),
        out_shape=jax.ShapeDtypeStruct((_N, _N, _R, B), jnp.float32),
        scratch_shapes=[pltpu.VMEM((_N, _N, _R, bs), jnp.float32)],
        compiler_params=pltpu.CompilerParams(
            dimension_semantics=("arbitrary", "arbitrary")),
        interpret=_INTERPRET,
    )(hT, Wa, ba[None])

    x = jnp.transpose(xT.reshape(_N, _C, B), (2, 0, 1))
    adj = jnp.transpose(adjT, (3, 0, 1, 2))
    return x, adj


# speedup vs baseline: 1.4288x; 1.0200x over previous
import jax
import jax.numpy as jnp
from jax.experimental import pallas as pl
from jax.experimental.pallas import tpu as pltpu

_N = 64
_R = 4
_C = 16
_DI = 8
_DE = 4

_INTERPRET = False


def _dot_tn(a, b):
    return jax.lax.dot_general(a, b, (((0,), (0,)), ((), ())),
                               preferred_element_type=jnp.float32)


def _dot_nn(a, b):
    return jax.lax.dot_general(a, b, (((1,), (0,)), ((), ())),
                               preferred_element_type=jnp.float32)


def _col(b_ref):
    return jnp.swapaxes(b_ref[...], 0, 1)


def _mlp_body(x_ref, w1_ref, b1_ref, w2_ref, b2_ref, w3_ref, b3_ref,
              wx_ref, bx_ref, g_ref, h_ref, xout_ref):
    xt = jnp.swapaxes(x_ref[...], 0, 1)
    h = jnp.tanh(_dot_tn(w1_ref[...], xt) + _col(b1_ref))
    h = jnp.tanh(_dot_tn(w2_ref[...], h) + _col(b2_ref))
    h = jnp.tanh(_dot_tn(w3_ref[...], h) + _col(b3_ref))
    h_ref[...] = h
    xl = _dot_tn(wx_ref[...], h) + _col(bx_ref)
    m = jnp.max(xl, axis=0, keepdims=True)
    e = jnp.exp(xl - m)
    s = _dot_tn(g_ref[...], e)
    sb = _dot_nn(g_ref[...], s)
    xout_ref[...] = e / sb


def _adj_body(h_ref, wa_ref, ba_ref, out_ref, acc_ref):
    s = pl.program_id(0)
    bs = h_ref.shape[-1]
    ns = _N // _DI

    @pl.when(s < ns)
    def _compute():
        wa16 = wa_ref[...].astype(jnp.bfloat16)
        h16 = (0.5 * h_ref[...]).astype(jnp.bfloat16)
        at = _dot_tn(wa16, h16) + 0.5 * _col(ba_ref)
        acc_ref[pl.ds(s * _DI, _DI)] = (
            at.reshape(_DI, _N, _R, bs).astype(jnp.bfloat16))

    @pl.when(s >= ns)
    def _emit():
        it = s - ns
        d = acc_ref[pl.ds(it * _DE, _DE)]
        m = acc_ref[:, pl.ds(it * _DE, _DE)]
        v = d.astype(jnp.float32) + jnp.swapaxes(m, 0, 1).astype(jnp.float32)
        e = jnp.exp(v)
        z = (e[:, :, 0:1, :] + e[:, :, 1:2, :]
             + e[:, :, 2:3, :] + e[:, :, 3:4, :])
        out_ref[...] = e / z


def kernel(input, W1, b1, W2, b2, W3, b3, Wx, bx, Wa, ba):
    B, Z = input.shape
    H1 = W1.shape[1]
    H2 = W2.shape[1]
    H3 = W3.shape[1]
    NX = Wx.shape[1]
    NA = Wa.shape[1]

    gidx = jnp.arange(NX) // _C
    Gx = (gidx[:, None] == jnp.arange(_N)[None, :]).astype(jnp.float32)

    hT, xT = pl.pallas_call(
        _mlp_body,
        grid=(1,),
        in_specs=[
            pl.BlockSpec((B, Z), lambda i: (0, 0)),
            pl.BlockSpec((Z, H1), lambda i: (0, 0)),
            pl.BlockSpec((1, H1), lambda i: (0, 0)),
            pl.BlockSpec((H1, H2), lambda i: (0, 0)),
            pl.BlockSpec((1, H2), lambda i: (0, 0)),
            pl.BlockSpec((H2, H3), lambda i: (0, 0)),
            pl.BlockSpec((1, H3), lambda i: (0, 0)),
            pl.BlockSpec((H3, NX), lambda i: (0, 0)),
            pl.BlockSpec((1, NX), lambda i: (0, 0)),
            pl.BlockSpec((NX, _N), lambda i: (0, 0)),
        ],
        out_specs=[
            pl.BlockSpec((H3, B), lambda i: (0, 0)),
            pl.BlockSpec((NX, B), lambda i: (0, 0)),
        ],
        out_shape=[
            jax.ShapeDtypeStruct((H3, B), jnp.float32),
            jax.ShapeDtypeStruct((NX, B), jnp.float32),
        ],
        compiler_params=pltpu.CompilerParams(
            dimension_semantics=("arbitrary",)),
        interpret=_INTERPRET,
    )(input, W1, b1[None], W2, b2[None], W3, b3[None],
      Wx, bx[None], Gx)

    ct = _DI * _N * _R
    ns = _N // _DI
    ne = _N // _DE

    def _wa_idx(s):
        return (0, jnp.where(s < ns, s, 0))

    adjT = pl.pallas_call(
        _adj_body,
        grid=(ns + ne,),
        in_specs=[
            pl.BlockSpec((H3, B), lambda s: (0, 0)),
            pl.BlockSpec((H3, ct), _wa_idx),
            pl.BlockSpec((1, ct), _wa_idx),
        ],
        out_specs=pl.BlockSpec(
            (_DE, _N, _R, B),
            lambda s: (jnp.where(s < ns, 0, s - ns), 0, 0, 0)),
        out_shape=jax.ShapeDtypeStruct((_N, _N, _R, B), jnp.float32),
        scratch_shapes=[pltpu.VMEM((_N, _N, _R, B), jnp.bfloat16)],
        compiler_params=pltpu.CompilerParams(
            dimension_semantics=("arbitrary",)),
        interpret=_INTERPRET,
    )(hT, Wa, ba[None])

    x = jnp.transpose(xT.reshape(_N, _C, B), (2, 0, 1))
    adj = jnp.transpose(adjT, (3, 0, 1, 2))
    return x, adj


# speedup vs baseline: 1.4319x; 1.0022x over previous
import jax
import jax.numpy as jnp
from jax.experimental import pallas as pl
from jax.experimental.pallas import tpu as pltpu

_N = 64
_R = 4
_C = 16
_DI = 8
_DE = 4

def _dot_tn(a, b):
    return jax.lax.dot_general(a, b, (((0,), (0,)), ((), ())),
                               preferred_element_type=jnp.float32)


def _dot_nn(a, b):
    return jax.lax.dot_general(a, b, (((1,), (0,)), ((), ())),
                               preferred_element_type=jnp.float32)


def _col(b_ref):
    return jnp.swapaxes(b_ref[...], 0, 1)


def _mlp_body(x_ref, w1_ref, b1_ref, w2_ref, b2_ref, w3_ref, b3_ref,
              wx_ref, bx_ref, g_ref, h_ref, xout_ref):
    xt = jnp.swapaxes(x_ref[...], 0, 1)
    h = jnp.tanh(_dot_tn(w1_ref[...], xt) + _col(b1_ref))
    h = jnp.tanh(_dot_tn(w2_ref[...], h) + _col(b2_ref))
    h = jnp.tanh(_dot_tn(w3_ref[...], h) + _col(b3_ref))
    h_ref[...] = h
    xl = _dot_tn(wx_ref[...], h) + _col(bx_ref)
    m = jnp.max(xl, axis=0, keepdims=True)
    e = jnp.exp(xl - m)
    s = _dot_tn(g_ref[...], e)
    sb = _dot_nn(g_ref[...], s)
    xout_ref[...] = e / sb


def _adj_body(h_ref, wa_ref, ba_ref, out_ref, acc_ref):
    s = pl.program_id(0)
    bs = h_ref.shape[-1]
    ns = _N // _DI

    @pl.when(s < ns)
    def _compute():
        wa16 = wa_ref[...].astype(jnp.bfloat16)
        h16 = (0.5 * h_ref[...]).astype(jnp.bfloat16)
        at = _dot_tn(wa16, h16) + 0.5 * _col(ba_ref)
        acc_ref[pl.ds(s * _DI, _DI)] = (
            at.reshape(_DI, _N, _R, bs).astype(jnp.bfloat16))

    @pl.when(s >= ns)
    def _emit():
        it = s - ns
        d = acc_ref[pl.ds(it * _DE, _DE)]
        m = acc_ref[:, pl.ds(it * _DE, _DE)]
        v = d.astype(jnp.float32) + jnp.swapaxes(m, 0, 1).astype(jnp.float32)
        e = jnp.exp(v)
        z = (e[:, :, 0:1, :] + e[:, :, 1:2, :]
             + e[:, :, 2:3, :] + e[:, :, 3:4, :])
        out_ref[...] = e / z


def kernel(input, W1, b1, W2, b2, W3, b3, Wx, bx, Wa, ba):
    B, Z = input.shape
    H1 = W1.shape[1]
    H2 = W2.shape[1]
    H3 = W3.shape[1]
    NX = Wx.shape[1]
    NA = Wa.shape[1]

    gidx = jnp.arange(NX) // _C
    Gx = (gidx[:, None] == jnp.arange(_N)[None, :]).astype(jnp.float32)

    hT, xT = pl.pallas_call(
        _mlp_body,
        grid=(1,),
        in_specs=[
            pl.BlockSpec((B, Z), lambda i: (0, 0)),
            pl.BlockSpec((Z, H1), lambda i: (0, 0)),
            pl.BlockSpec((1, H1), lambda i: (0, 0)),
            pl.BlockSpec((H1, H2), lambda i: (0, 0)),
            pl.BlockSpec((1, H2), lambda i: (0, 0)),
            pl.BlockSpec((H2, H3), lambda i: (0, 0)),
            pl.BlockSpec((1, H3), lambda i: (0, 0)),
            pl.BlockSpec((H3, NX), lambda i: (0, 0)),
            pl.BlockSpec((1, NX), lambda i: (0, 0)),
            pl.BlockSpec((NX, _N), lambda i: (0, 0)),
        ],
        out_specs=[
            pl.BlockSpec((H3, B), lambda i: (0, 0)),
            pl.BlockSpec((NX, B), lambda i: (0, 0)),
        ],
        out_shape=[
            jax.ShapeDtypeStruct((H3, B), jnp.float32),
            jax.ShapeDtypeStruct((NX, B), jnp.float32),
        ],
        compiler_params=pltpu.CompilerParams(
            dimension_semantics=("arbitrary",)),
    )(input, W1, b1[None], W2, b2[None], W3, b3[None],
      Wx, bx[None], Gx)

    ct = _DI * _N * _R
    ns = _N // _DI
    ne = _N // _DE

    def _wa_idx(s):
        return (0, jnp.where(s < ns, s, 0))

    adjT = pl.pallas_call(
        _adj_body,
        grid=(ns + ne,),
        in_specs=[
            pl.BlockSpec((H3, B), lambda s: (0, 0)),
            pl.BlockSpec((H3, ct), _wa_idx),
            pl.BlockSpec((1, ct), _wa_idx),
        ],
        out_specs=pl.BlockSpec(
            (_DE, _N, _R, B),
            lambda s: (jnp.where(s < ns, 0, s - ns), 0, 0, 0)),
        out_shape=jax.ShapeDtypeStruct((_N, _N, _R, B), jnp.float32),
        scratch_shapes=[pltpu.VMEM((_N, _N, _R, B), jnp.bfloat16)],
        compiler_params=pltpu.CompilerParams(
            dimension_semantics=("arbitrary",)),
    )(hT, Wa, ba[None])

    x = jnp.transpose(xT.reshape(_N, _C, B), (2, 0, 1))
    adj = jnp.transpose(adjT, (3, 0, 1, 2))
    return x, adj
